# Initial kernel scaffold; baseline (speedup 1.0000x reference)
import jax
import jax.numpy as jnp
from jax.experimental import pallas as pl


def kernel(x, Wi, Wp, q_in, q_pn, k_in, k_pn, v_in, v_pn):
    raise NotImplementedError("write your pallas kernel here")


# speedup vs baseline: 2.3322x; 2.3322x over previous
import jax
import jax.numpy as jnp
from jax import lax
from jax.experimental import pallas as pl

S = 2048
D = 768
R = 192
NI = 8
NP = 32
K = 3
TB = 256


def _body(x_ref, wr_ref, in_ref, pn_ref, out_ref):
    x = x_ref[...]
    scores = lax.dot_general(x, wr_ref[...], (((1,), (0,)), ((), ())),
                             preferred_element_type=jnp.float32)
    si = scores[:, :NI]
    sp = scores[:, NI:]
    si = si - jnp.max(si, axis=-1, keepdims=True)
    e = jnp.exp(si)
    w = e / jnp.sum(e, axis=-1, keepdims=True)

    proj = lax.dot_general(x, in_ref[0], (((1,), (0,)), ((), ())),
                           preferred_element_type=jnp.float32)
    xr = w[:, 0:1] * proj[:, 0:R]
    for n in range(1, NI):
        xr = xr + w[:, n:n + 1] * proj[:, n * R:(n + 1) * R]

    pn = pn_ref[0]
    pn_n = pn * lax.rsqrt(jnp.sum(pn * pn, axis=-1, keepdims=True) + 1e-8)

    iota = lax.broadcasted_iota(jnp.int32, (TB, NP), 1)
    for _ in range(K):
        m = jnp.max(sp, axis=-1, keepdims=True)
        cand = jnp.where(sp == m, iota, NP)
        amin = jnp.min(cand, axis=-1, keepdims=True)
        oh = iota == amin
        sel = lax.dot_general(oh.astype(jnp.float32), pn_n,
                              (((1,), (0,)), ((), ())),
                              preferred_element_type=jnp.float32,
                              precision=lax.Precision.HIGHEST)
        vtx = jnp.sum(xr * sel, axis=-1, keepdims=True)
        xr = xr - 2.0 * sel * vtx
        sp = jnp.where(oh, -jnp.inf, sp)

    out_ref[0] = xr


def kernel(x, Wi, Wp, q_in, q_pn, k_in, k_pn, v_in, v_pn):
    x2 = x.reshape(S, D)
    wr = jnp.concatenate([Wi.T, Wp.T], axis=1)
    instk = jnp.stack([q_in, k_in, v_in])
    instk = instk.transpose(0, 2, 1, 3).reshape(3, D, NI * R)
    pnstk = jnp.stack([q_pn, k_pn, v_pn])
    out = pl.pallas_call(
        _body,
        grid=(3, S // TB),
        in_specs=[
            pl.BlockSpec((TB, D), lambda c, t: (t, 0)),
            pl.BlockSpec((D, NI + NP), lambda c, t: (0, 0)),
            pl.BlockSpec((1, D, NI * R), lambda c, t: (c, 0, 0)),
            pl.BlockSpec((1, NP, R), lambda c, t: (c, 0, 0)),
        ],
        out_specs=pl.BlockSpec((1, TB, R), lambda c, t: (c, t, 0)),
        out_shape=jax.ShapeDtypeStruct((3, S, R), jnp.float32),
    )(x2, wr, instk, pnstk)
    return (out[0].reshape(1, S, R), out[1].reshape(1, S, R),
            out[2].reshape(1, S, R))


# speedup vs baseline: 2.5164x; 1.0790x over previous
import jax
import jax.numpy as jnp
from jax import lax
from jax.experimental import pallas as pl

S = 2048
D = 768
R = 192
NI = 8
NP = 32
K = 3
TB = 256


def _body(x_ref, wr_ref, in_ref, pn_ref, out_ref):
    x = x_ref[...]
    scores = lax.dot_general(x, wr_ref[...], (((1,), (0,)), ((), ())),
                             preferred_element_type=jnp.float32)
    si = scores[:, :NI]
    sp = scores[:, NI:]
    si = si - jnp.max(si, axis=-1, keepdims=True)
    e = jnp.exp(si)
    w = e / jnp.sum(e, axis=-1, keepdims=True)

    proj = lax.dot_general(x.astype(jnp.bfloat16), in_ref[0],
                           (((1,), (0,)), ((), ())),
                           preferred_element_type=jnp.float32)
    xr = w[:, 0:1] * proj[:, 0:R]
    for n in range(1, NI):
        xr = xr + w[:, n:n + 1] * proj[:, n * R:(n + 1) * R]

    pn = pn_ref[0]
    pn_n = pn * lax.rsqrt(jnp.sum(pn * pn, axis=-1, keepdims=True) + 1e-8)

    iota = lax.broadcasted_iota(jnp.int32, (TB, NP), 1)
    for _ in range(K):
        m = jnp.max(sp, axis=-1, keepdims=True)
        cand = jnp.where(sp == m, iota, NP)
        amin = jnp.min(cand, axis=-1, keepdims=True)
        oh = iota == amin
        sel = lax.dot_general(oh.astype(jnp.float32), pn_n,
                              (((1,), (0,)), ((), ())),
                              preferred_element_type=jnp.float32,
                              precision=lax.Precision.HIGHEST)
        vtx = jnp.sum(xr * sel, axis=-1, keepdims=True)
        xr = xr - 2.0 * sel * vtx
        sp = jnp.where(oh, -jnp.inf, sp)

    out_ref[0] = xr


def kernel(x, Wi, Wp, q_in, q_pn, k_in, k_pn, v_in, v_pn):
    x2 = x.reshape(S, D)
    wr = jnp.concatenate([Wi.T, Wp.T], axis=1)
    instk = jnp.stack([q_in, k_in, v_in])
    instk = instk.transpose(0, 2, 1, 3).reshape(3, D, NI * R)
    instk = instk.astype(jnp.bfloat16)
    pnstk = jnp.stack([q_pn, k_pn, v_pn])
    out = pl.pallas_call(
        _body,
        grid=(3, S // TB),
        in_specs=[
            pl.BlockSpec((TB, D), lambda c, t: (t, 0)),
            pl.BlockSpec((D, NI + NP), lambda c, t: (0, 0)),
            pl.BlockSpec((1, D, NI * R), lambda c, t: (c, 0, 0)),
            pl.BlockSpec((1, NP, R), lambda c, t: (c, 0, 0)),
        ],
        out_specs=pl.BlockSpec((1, TB, R), lambda c, t: (c, t, 0)),
        out_shape=jax.ShapeDtypeStruct((3, S, R), jnp.float32),
    )(x2, wr, instk, pnstk)
    return (out[0].reshape(1, S, R), out[1].reshape(1, S, R),
            out[2].reshape(1, S, R))


# speedup vs baseline: 2.6826x; 1.0660x over previous
import jax
import jax.numpy as jnp
from jax import lax
from jax.experimental import pallas as pl

S = 2048
D = 768
R = 192
RP = 256
NI = 8
NP = 32
K = 3
TB = 256


def _body(x_ref, wr_ref, in_ref, pn_ref, out_ref):
    x = x_ref[...]
    scores = lax.dot_general(x, wr_ref[...], (((1,), (0,)), ((), ())),
                             preferred_element_type=jnp.float32)
    si = scores[:, :NI]
    sp = scores[:, NI:]
    si = si - jnp.max(si, axis=-1, keepdims=True)
    e = jnp.exp(si)
    w = e / jnp.sum(e, axis=-1, keepdims=True)

    proj = lax.dot_general(x.astype(jnp.bfloat16), in_ref[0],
                           (((1,), (0,)), ((), ())),
                           preferred_element_type=jnp.float32)
    xr = w[:, 0:1] * proj[:, 0:RP]
    for n in range(1, NI):
        xr = xr + w[:, n:n + 1] * proj[:, n * RP:(n + 1) * RP]

    pn = pn_ref[0]
    pn_n = pn * lax.rsqrt(jnp.sum(pn * pn, axis=-1, keepdims=True) + 1e-8)

    iota = lax.broadcasted_iota(jnp.int32, (TB, NP), 1)
    for _ in range(K):
        m = jnp.max(sp, axis=-1, keepdims=True)
        cand = jnp.where(sp == m, iota, NP)
        amin = jnp.min(cand, axis=-1, keepdims=True)
        oh = iota == amin
        sel = lax.dot_general(oh.astype(jnp.float32), pn_n,
                              (((1,), (0,)), ((), ())),
                              preferred_element_type=jnp.float32)
        vtx = jnp.sum(xr * sel, axis=-1, keepdims=True)
        xr = xr - 2.0 * sel * vtx
        sp = jnp.where(oh, -jnp.inf, sp)

    out_ref[0] = xr[:, :R]


def kernel(x, Wi, Wp, q_in, q_pn, k_in, k_pn, v_in, v_pn):
    x2 = x.reshape(S, D)
    wr = jnp.concatenate([Wi.T, Wp.T], axis=1)
    instk = jnp.stack([q_in, k_in, v_in])
    instk = jnp.pad(instk, ((0, 0), (0, 0), (0, 0), (0, RP - R)))
    instk = instk.transpose(0, 2, 1, 3).reshape(3, D, NI * RP)
    instk = instk.astype(jnp.bfloat16)
    pnstk = jnp.stack([q_pn, k_pn, v_pn])
    pnstk = jnp.pad(pnstk, ((0, 0), (0, 0), (0, RP - R)))
    out = pl.pallas_call(
        _body,
        grid=(3, S // TB),
        in_specs=[
            pl.BlockSpec((TB, D), lambda c, t: (t, 0)),
            pl.BlockSpec((D, NI + NP), lambda c, t: (0, 0)),
            pl.BlockSpec((1, D, NI * RP), lambda c, t: (c, 0, 0)),
            pl.BlockSpec((1, NP, RP), lambda c, t: (c, 0, 0)),
        ],
        out_specs=pl.BlockSpec((1, TB, R), lambda c, t: (c, t, 0)),
        out_shape=jax.ShapeDtypeStruct((3, S, R), jnp.float32),
    )(x2, wr, instk, pnstk)
    return (out[0].reshape(1, S, R), out[1].reshape(1, S, R),
            out[2].reshape(1, S, R))


# speedup vs baseline: 3.7794x; 1.4089x over previous
import jax
import jax.numpy as jnp
from jax import lax
from jax.experimental import pallas as pl

S = 2048
D = 768
R = 192
RP = 256
NI = 8
NP = 32
K = 3
NC = 3
TB = 256


def _body(x_ref, wr_ref, in_ref, pn_ref, out_ref):
    x = x_ref[...]
    scores = lax.dot_general(x, wr_ref[...], (((1,), (0,)), ((), ())),
                             preferred_element_type=jnp.float32)
    si = scores[:, :NI]
    sp = scores[:, NI:]
    si = si - jnp.max(si, axis=-1, keepdims=True)
    e = jnp.exp(si)
    w = e / jnp.sum(e, axis=-1, keepdims=True)

    proj = lax.dot_general(x.astype(jnp.bfloat16), in_ref[...],
                           (((1,), (0,)), ((), ())),
                           preferred_element_type=jnp.float32)
    xrs = []
    for c in range(NC):
        xr = w[:, 0:1] * proj[:, c * NI * RP:c * NI * RP + RP]
        for n in range(1, NI):
            base = (c * NI + n) * RP
            xr = xr + w[:, n:n + 1] * proj[:, base:base + RP]
        xrs.append(xr)

    pn = pn_ref[...]
    blocks = []
    for c in range(NC):
        blk = pn[:, c * RP:(c + 1) * RP]
        nrm = lax.rsqrt(jnp.sum(blk * blk, axis=-1, keepdims=True) + 1e-8)
        blocks.append(blk * nrm)
    pn_n = jnp.concatenate(blocks, axis=1)

    iota = lax.broadcasted_iota(jnp.int32, (TB, NP), 1)
    for _ in range(K):
        m = jnp.max(sp, axis=-1, keepdims=True)
        cand = jnp.where(sp == m, iota, NP)
        amin = jnp.min(cand, axis=-1, keepdims=True)
        oh = iota == amin
        sel = lax.dot_general(oh.astype(jnp.float32), pn_n,
                              (((1,), (0,)), ((), ())),
                              preferred_element_type=jnp.float32)
        for c in range(NC):
            sc = sel[:, c * RP:(c + 1) * RP]
            vtx = jnp.sum(xrs[c] * sc, axis=-1, keepdims=True)
            xrs[c] = xrs[c] - 2.0 * sc * vtx
        sp = jnp.where(oh, -jnp.inf, sp)

    for c in range(NC):
        out_ref[:, c * RP:c * RP + RP] = xrs[c]


def kernel(x, Wi, Wp, q_in, q_pn, k_in, k_pn, v_in, v_pn):
    x2 = x.reshape(S, D)
    wr = jnp.concatenate([Wi.T, Wp.T], axis=1)
    instk = jnp.stack([q_in, k_in, v_in])
    instk = jnp.pad(instk, ((0, 0), (0, 0), (0, 0), (0, RP - R)))
    instk = instk.transpose(2, 0, 1, 3).reshape(D, NC * NI * RP)
    instk = instk.astype(jnp.bfloat16)
    pnstk = jnp.stack([q_pn, k_pn, v_pn])
    pnstk = jnp.pad(pnstk, ((0, 0), (0, 0), (0, RP - R)))
    pnstk = pnstk.transpose(1, 0, 2).reshape(NP, NC * RP)
    out = pl.pallas_call(
        _body,
        grid=(S // TB,),
        in_specs=[
            pl.BlockSpec((TB, D), lambda t: (t, 0)),
            pl.BlockSpec((D, NI + NP), lambda t: (0, 0)),
            pl.BlockSpec((D, NC * NI * RP), lambda t: (0, 0)),
            pl.BlockSpec((NP, NC * RP), lambda t: (0, 0)),
        ],
        out_specs=pl.BlockSpec((TB, NC * RP), lambda t: (t, 0)),
        out_shape=jax.ShapeDtypeStruct((S, NC * RP), jnp.float32),
    )(x2, wr, instk, pnstk)
    return (out[:, 0:R].reshape(1, S, R),
            out[:, RP:RP + R].reshape(1, S, R),
            out[:, 2 * RP:2 * RP + R].reshape(1, S, R))
